# baseline (speedup 1.0000x reference)
import functools

import jax
import jax.numpy as jnp
from jax import lax
from jax.experimental import pallas as pl
from jax.experimental.pallas import tpu as pltpu
from jax.experimental.pallas import tpu_sc as plsc

NTOKEN = 100000
EMB_DIM = 64
OUT_DIM = 2 * EMB_DIM
BATCH = 4096
SEQ = 20
TOT = BATCH * SEQ

NUM_CORES = 2
NUM_SUBCORES = 16
NW = NUM_CORES * NUM_SUBCORES
BPW = TOT // NW
CHUNK = 320
NCHUNK = BPW // CHUNK


@functools.partial(
    pl.kernel,
    mesh=plsc.VectorSubcoreMesh(core_axis_name="c", subcore_axis_name="s"),
    out_type=jax.ShapeDtypeStruct((TOT, OUT_DIM), jnp.float32),
    scratch_types=[
        pltpu.VMEM((BPW,), jnp.int32),
        pltpu.VMEM((CHUNK, OUT_DIM), jnp.float32),
        pltpu.VMEM((CHUNK, OUT_DIM), jnp.float32),
        pltpu.SemaphoreType.DMA,
        pltpu.SemaphoreType.DMA,
        pltpu.SemaphoreType.DMA,
        pltpu.SemaphoreType.DMA,
        pltpu.SemaphoreType.DMA,
        pltpu.SemaphoreType.DMA,
    ],
    compiler_params=pltpu.CompilerParams(use_tc_tiling_on_sc=False),
)
def _emb_lookup(a_hbm, b_hbm, x_hbm, out_hbm, idx_v, r0, r1,
                sa0, sa1, sb0, sb1, sw0, sw1):
    wid = lax.axis_index("s") * NUM_CORES + lax.axis_index("c")
    pltpu.sync_copy(x_hbm.at[wid], idx_v)
    rows = (r0, r1)
    sa = (sa0, sa1)
    sb = (sb0, sb1)
    sw = (sw0, sw1)
    ga = [None, None]
    gb = [None, None]
    wr = [None, None]

    def idx(j):
        return idx_v.at[pl.ds(j * CHUNK, CHUNK)]

    ga[0] = pltpu.async_copy(a_hbm.at[idx(0)], rows[0], sa[0])
    for j in range(NCHUNK):
        cur = j % 2
        ga[cur].wait()
        gb[cur] = pltpu.async_copy(b_hbm.at[idx(j)], rows[cur], sb[cur],
                                   add=True)
        if j + 1 < NCHUNK:
            nxt = (j + 1) % 2
            if wr[nxt] is not None:
                wr[nxt].wait()
            ga[nxt] = pltpu.async_copy(a_hbm.at[idx(j + 1)], rows[nxt],
                                       sa[nxt])
        gb[cur].wait()
        base = wid * BPW + j * CHUNK
        wr[cur] = pltpu.async_copy(rows[cur], out_hbm.at[pl.ds(base, CHUNK)],
                                   sw[cur])
    for w in wr:
        if w is not None:
            w.wait()


def kernel(x, emb_w, embc_w):
    pad_a = jnp.pad(emb_w, ((0, 0), (0, EMB_DIM)))
    pad_b = jnp.pad(embc_w, ((0, 0), (EMB_DIM, 0)))
    xt = x.T.reshape(NW, BPW)
    out = _emb_lookup(pad_a, pad_b, xt)
    out = out.reshape(SEQ, BATCH, OUT_DIM)
    return out.transpose(1, 0, 2)
